# Initial kernel scaffold; baseline (speedup 1.0000x reference)
import jax
import jax.numpy as jnp
from jax.experimental import pallas as pl


def kernel(node_data, edge_data, edge_index, lap_eigvec, emb_table, lap_W, order_table):
    raise NotImplementedError("write your pallas kernel here")


# speedup vs baseline: 44.8479x; 44.8479x over previous
import functools

import jax
import jax.numpy as jnp
from jax import lax
from jax.experimental import pallas as pl
from jax.experimental.pallas import tpu as pltpu
from jax.experimental.pallas import tpu_sc as plsc

_B, _N, _E, _F, _D, _K, _V = 8, 1024, 3072, 4, 256, 16, 8192
_T = _N + _E
_BN = _B * _N
_L = 16
_C = 32
_TOK_PER_W = 1024
_NCHUNK = _TOK_PER_W // _C
_RB = 512


def _tbl_body(eig_ref, w_ref, o_ref, out_ref):
    x = eig_ref[...]
    w0 = w_ref[0:_K, :]
    w1 = w_ref[_K:2 * _K, :]
    p0 = jnp.dot(x, w0, preferred_element_type=jnp.float32)
    p1 = jnp.dot(x, w1, preferred_element_type=jnp.float32)
    o0 = o_ref[0:1, :]
    o1 = o_ref[1:2, :]
    out_ref[0] = p0
    out_ref[1] = p1 + o0
    out_ref[2] = p0 + p1 + o1
    out_ref[3] = p1 + o1


def _build_table(lap_eigvec, lap_W, order_table):
    out = pl.pallas_call(
        _tbl_body,
        grid=(_BN // _RB,),
        in_specs=[
            pl.BlockSpec((_RB, _K), lambda i: (i, 0)),
            pl.BlockSpec((2 * _K, _D), lambda i: (0, 0)),
            pl.BlockSpec((2, _D), lambda i: (0, 0)),
        ],
        out_specs=pl.BlockSpec((4, _RB, _D), lambda i: (0, i, 0)),
        out_shape=jax.ShapeDtypeStruct((4, _BN, _D), jnp.float32),
    )(lap_eigvec, lap_W, order_table)
    return out.reshape(4 * _BN, _D)


def _sc_body(nd_ref, ed_ref, src_ref, dst_ref, emb_ref, tb_ref, out_ref,
             ia, ib, ic, buf_a, buf_b, buf_c, outv, sem_a, sem_b, sem_c):
    cid = lax.axis_index("c")
    sid = lax.axis_index("s")
    wid = sid * 2 + cid

    def sum_rows(with_c):
        def body(c, carry):
            for h in range(_D // _L):
                sl = pl.ds(h * _L, _L)
                acc = (buf_a[4 * c, sl] + buf_a[4 * c + 1, sl]
                       + buf_a[4 * c + 2, sl] + buf_a[4 * c + 3, sl])
                acc = acc + buf_b[c, sl]
                if with_c:
                    acc = acc + buf_c[c, sl]
                outv[c, sl] = acc
            return carry
        lax.fori_loop(0, _C, body, None)

    def edge_chunk(i, b, j):
        ebase = b * _E + j * _TOK_PER_W + i * _C
        obase = b * _T + _N + j * _TOK_PER_W + i * _C
        pltpu.sync_copy(ed_ref.at[pl.ds(4 * ebase, 4 * _C)], ia)
        pltpu.sync_copy(src_ref.at[pl.ds(ebase, _C)], ib)
        pltpu.sync_copy(dst_ref.at[pl.ds(ebase, _C)], ic)
        for k in range(_C // _L):
            sl = pl.ds(k * _L, _L)
            sv = ib[sl]
            dv = ic[sl]
            ib[sl] = sv + b * _N
            ic[sl] = (dv + (_BN + b * _N)
                      + jnp.where(sv == dv, 2 * _BN, 0))
        cp_a = pltpu.async_copy(emb_ref.at[ia], buf_a, sem_a)
        cp_b = pltpu.async_copy(tb_ref.at[ib], buf_b, sem_b)
        cp_c = pltpu.async_copy(tb_ref.at[ic], buf_c, sem_c)
        cp_a.wait()
        cp_b.wait()
        cp_c.wait()
        sum_rows(with_c=True)
        pltpu.sync_copy(outv, out_ref.at[pl.ds(obase, _C)])

    def node_chunk(i, b):
        gbase = b * _N + i * _C
        obase = b * _T + i * _C
        pltpu.sync_copy(nd_ref.at[pl.ds(4 * gbase, 4 * _C)], ia)
        cp_a = pltpu.async_copy(emb_ref.at[ia], buf_a, sem_a)
        cp_b = pltpu.async_copy(tb_ref.at[pl.ds(2 * _BN + gbase, _C)],
                                buf_b, sem_b)
        cp_a.wait()
        cp_b.wait()
        sum_rows(with_c=False)
        pltpu.sync_copy(outv, out_ref.at[pl.ds(obase, _C)])

    @pl.when(wid < _B)
    def _():
        b = wid

        def body(i, carry):
            node_chunk(i, b)
            return carry
        lax.fori_loop(0, _NCHUNK, body, None)

    @pl.when(wid >= _B)
    def _():
        ew = wid - _B
        b = ew // 3
        j = ew % 3

        def body(i, carry):
            edge_chunk(i, b, j)
            return carry
        lax.fori_loop(0, _NCHUNK, body, None)


def _gather_sum(nd_flat, ed_flat, src, dst, emb_table, tbig):
    mesh = plsc.VectorSubcoreMesh(core_axis_name="c", subcore_axis_name="s")
    fn = pl.kernel(
        _sc_body,
        out_type=jax.ShapeDtypeStruct((_B * _T, _D), jnp.float32),
        mesh=mesh,
        scratch_types=[
            pltpu.VMEM((4 * _C,), jnp.int32),
            pltpu.VMEM((_C,), jnp.int32),
            pltpu.VMEM((_C,), jnp.int32),
            pltpu.VMEM((4 * _C, _D), jnp.float32),
            pltpu.VMEM((_C, _D), jnp.float32),
            pltpu.VMEM((_C, _D), jnp.float32),
            pltpu.VMEM((_C, _D), jnp.float32),
            pltpu.SemaphoreType.DMA,
            pltpu.SemaphoreType.DMA,
            pltpu.SemaphoreType.DMA,
        ],
    )
    return fn(nd_flat, ed_flat, src, dst, emb_table, tbig)


def kernel(node_data, edge_data, edge_index, lap_eigvec, emb_table, lap_W,
           order_table):
    tbig = _build_table(lap_eigvec, lap_W, order_table)
    feat = _gather_sum(node_data.reshape(-1), edge_data.reshape(-1),
                       edge_index[0], edge_index[1], emb_table, tbig)
    padded_feature = feat.reshape(_B, _T, _D)
    node_part = jnp.broadcast_to(
        jnp.arange(_N, dtype=edge_index.dtype)[None, :, None], (_B, _N, 2))
    edge_part = jnp.transpose(edge_index).reshape(_B, _E, 2)
    padded_index = jnp.concatenate([node_part, edge_part], axis=1)
    padding_mask = jnp.zeros((_B, _T), dtype=jnp.bool_)
    return padded_feature, padding_mask, padded_index


# speedup vs baseline: 69.9319x; 1.5593x over previous
import functools

import jax
import jax.numpy as jnp
from jax import lax
from jax.experimental import pallas as pl
from jax.experimental.pallas import tpu as pltpu
from jax.experimental.pallas import tpu_sc as plsc

_B, _N, _E, _F, _D, _K, _V = 8, 1024, 3072, 4, 256, 16, 8192
_T = _N + _E
_BN = _B * _N
_L = 16
_C = 32
_TOK_PER_W = 1024
_NCHUNK = _TOK_PER_W // _C
_NPAIR = _NCHUNK // 2
_RB = 512


def _tbl_body(eig_ref, w_ref, o_ref, out_ref):
    x = eig_ref[...]
    w0 = w_ref[0:_K, :]
    w1 = w_ref[_K:2 * _K, :]
    p0 = jnp.dot(x, w0, preferred_element_type=jnp.float32)
    p1 = jnp.dot(x, w1, preferred_element_type=jnp.float32)
    o0 = o_ref[0:1, :]
    o1 = o_ref[1:2, :]
    out_ref[0] = p0
    out_ref[1] = p1 + o0
    out_ref[2] = p0 + p1 + o1
    out_ref[3] = p1 + o1


def _build_table(lap_eigvec, lap_W, order_table):
    out = pl.pallas_call(
        _tbl_body,
        grid=(_BN // _RB,),
        in_specs=[
            pl.BlockSpec((_RB, _K), lambda i: (i, 0)),
            pl.BlockSpec((2 * _K, _D), lambda i: (0, 0)),
            pl.BlockSpec((2, _D), lambda i: (0, 0)),
        ],
        out_specs=pl.BlockSpec((4, _RB, _D), lambda i: (0, i, 0)),
        out_shape=jax.ShapeDtypeStruct((4, _BN, _D), jnp.float32),
    )(lap_eigvec, lap_W, order_table)
    return out.reshape(4 * _BN, _D)


def _sc_body(nd_ref, ed_ref, src_ref, dst_ref, emb_ref, tb_ref, out_ref,
             ia_all, ib_all, ic_all,
             buf_a, buf_b, buf_c, outv,
             sem_i, sem_a, sem_b, sem_c, sem_o):
    cid = lax.axis_index("c")
    sid = lax.axis_index("s")
    wid = sid * 2 + cid

    def sum_rows(slot, with_c):
        def body(c, carry):
            for h in range(_D // _L):
                sl = pl.ds(h * _L, _L)
                acc = (buf_a[slot][4 * c, sl] + buf_a[slot][4 * c + 1, sl]
                       + buf_a[slot][4 * c + 2, sl] + buf_a[slot][4 * c + 3, sl])
                acc = acc + buf_b[slot][c, sl]
                if with_c:
                    acc = acc + buf_c[slot][c, sl]
                outv[slot][c, sl] = acc
            return carry
        lax.fori_loop(0, _C, body, None)

    def run(tok0, obase0, is_edge):
        def gathers(i, slot):
            cps = [pltpu.make_async_copy(
                emb_ref.at[ia_all.at[pl.ds(i * 4 * _C, 4 * _C)]],
                buf_a[slot], sem_a[slot])]
            if is_edge:
                cps.append(pltpu.make_async_copy(
                    tb_ref.at[ib_all.at[pl.ds(i * _C, _C)]],
                    buf_b[slot], sem_b[slot]))
                cps.append(pltpu.make_async_copy(
                    tb_ref.at[ic_all.at[pl.ds(i * _C, _C)]],
                    buf_c[slot], sem_c[slot]))
            else:
                cps.append(pltpu.make_async_copy(
                    tb_ref.at[pl.ds(2 * _BN + tok0 + i * _C, _C)],
                    buf_b[slot], sem_b[slot]))
            return cps

        def store(i, slot):
            return pltpu.make_async_copy(
                outv[slot], out_ref.at[pl.ds(obase0 + i * _C, _C)],
                sem_o[slot])

        def consume(i, slot, s):
            for cp in gathers(i, slot):
                cp.wait()

            @pl.when(s > 0)
            def _():
                store(i, slot).wait()
            sum_rows(slot, with_c=is_edge)
            store(i, slot).start()

        def fire(i, slot):
            for cp in gathers(i, slot):
                cp.start()

        fire(0, 0)

        def pair(s, carry):
            fire(2 * s + 1, 1)
            consume(2 * s, 0, s)

            @pl.when(s < _NPAIR - 1)
            def _():
                fire(2 * s + 2, 0)
            consume(2 * s + 1, 1, s)
            return carry

        lax.fori_loop(0, _NPAIR, pair, None)
        store(_NCHUNK - 2, 0).wait()
        store(_NCHUNK - 1, 1).wait()

    @pl.when(wid < _B)
    def _():
        b = wid
        tok0 = b * _N
        pltpu.async_copy(nd_ref.at[pl.ds(4 * tok0, 4 * _TOK_PER_W)],
                         ia_all, sem_i).wait()
        run(tok0, b * _T, is_edge=False)

    @pl.when(wid >= _B)
    def _():
        ew = wid - _B
        b = ew // 3
        j = ew % 3
        tok0 = b * _E + j * _TOK_PER_W
        cp1 = pltpu.async_copy(ed_ref.at[pl.ds(4 * tok0, 4 * _TOK_PER_W)],
                               ia_all, sem_i)
        cp2 = pltpu.async_copy(src_ref.at[pl.ds(tok0, _TOK_PER_W)],
                               ib_all, sem_i)
        cp3 = pltpu.async_copy(dst_ref.at[pl.ds(tok0, _TOK_PER_W)],
                               ic_all, sem_i)
        cp1.wait()
        cp2.wait()
        cp3.wait()

        def arith(k, carry):
            sl = pl.ds(k * _L, _L)
            sv = ib_all[sl]
            dv = ic_all[sl]
            ib_all[sl] = sv + b * _N
            ic_all[sl] = (dv + (_BN + b * _N)
                          + jnp.where(sv == dv, 2 * _BN, 0))
            return carry

        lax.fori_loop(0, _TOK_PER_W // _L, arith, None)
        run(tok0, b * _T + _N + j * _TOK_PER_W, is_edge=True)


def _gather_sum(nd_flat, ed_flat, src, dst, emb_table, tbig):
    mesh = plsc.VectorSubcoreMesh(core_axis_name="c", subcore_axis_name="s")
    fn = pl.kernel(
        _sc_body,
        out_type=jax.ShapeDtypeStruct((_B * _T, _D), jnp.float32),
        mesh=mesh,
        scratch_types=[
            pltpu.VMEM((4 * _TOK_PER_W,), jnp.int32),
            pltpu.VMEM((_TOK_PER_W,), jnp.int32),
            pltpu.VMEM((_TOK_PER_W,), jnp.int32),
            [pltpu.VMEM((4 * _C, _D), jnp.float32)] * 2,
            [pltpu.VMEM((_C, _D), jnp.float32)] * 2,
            [pltpu.VMEM((_C, _D), jnp.float32)] * 2,
            [pltpu.VMEM((_C, _D), jnp.float32)] * 2,
            pltpu.SemaphoreType.DMA,
            [pltpu.SemaphoreType.DMA] * 2,
            [pltpu.SemaphoreType.DMA] * 2,
            [pltpu.SemaphoreType.DMA] * 2,
            [pltpu.SemaphoreType.DMA] * 2,
        ],
    )
    return fn(nd_flat, ed_flat, src, dst, emb_table, tbig)


def kernel(node_data, edge_data, edge_index, lap_eigvec, emb_table, lap_W,
           order_table):
    tbig = _build_table(lap_eigvec, lap_W, order_table)
    feat = _gather_sum(node_data.reshape(-1), edge_data.reshape(-1),
                       edge_index[0], edge_index[1], emb_table, tbig)
    padded_feature = feat.reshape(_B, _T, _D)
    node_part = jnp.broadcast_to(
        jnp.arange(_N, dtype=edge_index.dtype)[None, :, None], (_B, _N, 2))
    edge_part = jnp.transpose(edge_index).reshape(_B, _E, 2)
    padded_index = jnp.concatenate([node_part, edge_part], axis=1)
    padding_mask = jnp.zeros((_B, _T), dtype=jnp.bool_)
    return padded_feature, padding_mask, padded_index


# speedup vs baseline: 72.4902x; 1.0366x over previous
import functools

import jax
import jax.numpy as jnp
from jax import lax
from jax.experimental import pallas as pl
from jax.experimental.pallas import tpu as pltpu
from jax.experimental.pallas import tpu_sc as plsc

_B, _N, _E, _F, _D, _K, _V = 8, 1024, 3072, 4, 256, 16, 8192
_T = _N + _E
_BN = _B * _N
_L = 16
_C = 32
_NNODE = _N // 4
_NEDGE = _E // 4
_NCH_N = _NNODE // _C
_NCHUNK = (_NNODE + _NEDGE) // _C
_NPAIR = _NCHUNK // 2
_RB = 1024


def _tbl_body(eig_ref, w_ref, o_ref, out_ref):
    x = eig_ref[...]
    w0 = w_ref[0:_K, :]
    w1 = w_ref[_K:2 * _K, :]
    p0 = jnp.dot(x, w0, preferred_element_type=jnp.float32)
    p1 = jnp.dot(x, w1, preferred_element_type=jnp.float32)
    o0 = o_ref[0:1, :]
    o1 = o_ref[1:2, :]
    out_ref[0] = p0
    out_ref[1] = p1 + o0
    out_ref[2] = p0 + p1 + o1
    out_ref[3] = p1 + o1


def _build_table(lap_eigvec, lap_W, order_table):
    out = pl.pallas_call(
        _tbl_body,
        grid=(_BN // _RB,),
        in_specs=[
            pl.BlockSpec((_RB, _K), lambda i: (i, 0)),
            pl.BlockSpec((2 * _K, _D), lambda i: (0, 0)),
            pl.BlockSpec((2, _D), lambda i: (0, 0)),
        ],
        out_specs=pl.BlockSpec((4, _RB, _D), lambda i: (0, i, 0)),
        out_shape=jax.ShapeDtypeStruct((4, _BN, _D), jnp.float32),
    )(lap_eigvec, lap_W, order_table)
    return out.reshape(4 * _BN, _D)


def _sc_body(nd_ref, ed_ref, src_ref, dst_ref, emb_ref, tb_ref, out_ref,
             ia_n, ia_e, ibc, itmp,
             buf_a, buf_bc, outv,
             sem_i, sem_a, sem_b, sem_o):
    cid = lax.axis_index("c")
    sid = lax.axis_index("s")
    wid = sid * 2 + cid
    b = wid // 4
    q = wid % 4

    ntok0 = b * _N + q * _NNODE
    etok0 = b * _E + q * _NEDGE
    onode0 = b * _T + q * _NNODE
    oedge0 = b * _T + _N + q * _NEDGE

    cp1 = pltpu.async_copy(nd_ref.at[pl.ds(4 * ntok0, 4 * _NNODE)],
                           ia_n, sem_i)
    cp2 = pltpu.async_copy(ed_ref.at[pl.ds(4 * etok0, 4 * _NEDGE)],
                           ia_e, sem_i)
    cp3 = pltpu.async_copy(src_ref.at[pl.ds(etok0, _NEDGE)],
                           itmp.at[pl.ds(0, _NEDGE)], sem_i)
    cp4 = pltpu.async_copy(dst_ref.at[pl.ds(etok0, _NEDGE)],
                           itmp.at[pl.ds(_NEDGE, _NEDGE)], sem_i)
    cp1.wait()
    cp2.wait()
    cp3.wait()
    cp4.wait()

    def arith(k, carry):
        sl = pl.ds(k * _L, _L)
        sv = itmp[sl]
        dv = itmp[pl.ds(_NEDGE + k * _L, _L)]
        i = k // 2
        h = k % 2
        ibc[pl.ds(2 * _C * i + h * _L, _L)] = sv + b * _N
        ibc[pl.ds(2 * _C * i + _C + h * _L, _L)] = (
            dv + (_BN + b * _N) + jnp.where(sv == dv, 2 * _BN, 0))
        return carry

    lax.fori_loop(0, _NEDGE // _L, arith, None)

    def gathers_node(i, slot):
        return [
            pltpu.make_async_copy(
                emb_ref.at[ia_n.at[pl.ds(i * 4 * _C, 4 * _C)]],
                buf_a[slot], sem_a[slot]),
            pltpu.make_async_copy(
                tb_ref.at[pl.ds(2 * _BN + ntok0 + i * _C, _C)],
                buf_bc[slot].at[pl.ds(0, _C)], sem_b[slot]),
        ]

    def gathers_edge(i, slot):
        e = i - _NCH_N
        return [
            pltpu.make_async_copy(
                emb_ref.at[ia_e.at[pl.ds(e * 4 * _C, 4 * _C)]],
                buf_a[slot], sem_a[slot]),
            pltpu.make_async_copy(
                tb_ref.at[ibc.at[pl.ds(e * 2 * _C, 2 * _C)]],
                buf_bc[slot], sem_b[slot]),
        ]

    def store(i, slot):
        obase = jnp.where(i < _NCH_N, onode0 + i * _C,
                          oedge0 + (i - _NCH_N) * _C)
        return pltpu.make_async_copy(
            outv[slot], out_ref.at[pl.ds(obase, _C)], sem_o[slot])

    def sum_rows(slot, with_c):
        def body(c, carry):
            for h in range(_D // _L):
                sl = pl.ds(h * _L, _L)
                acc = (buf_a[slot][4 * c, sl] + buf_a[slot][4 * c + 1, sl]
                       + buf_a[slot][4 * c + 2, sl] + buf_a[slot][4 * c + 3, sl])
                acc = acc + buf_bc[slot][c, sl]
                if with_c:
                    acc = acc + buf_bc[slot][_C + c, sl]
                outv[slot][c, sl] = acc
            return carry
        lax.fori_loop(0, _C, body, None)

    def fire(i, slot):
        @pl.when(i < _NCH_N)
        def _():
            for cp in gathers_node(i, slot):
                cp.start()

        @pl.when(i >= _NCH_N)
        def _():
            for cp in gathers_edge(i, slot):
                cp.start()

    def consume(i, slot, s):
        @pl.when(i < _NCH_N)
        def _():
            for cp in gathers_node(i, slot):
                cp.wait()

        @pl.when(i >= _NCH_N)
        def _():
            for cp in gathers_edge(i, slot):
                cp.wait()

        @pl.when(s > 0)
        def _():
            store(i, slot).wait()

        @pl.when(i < _NCH_N)
        def _():
            sum_rows(slot, with_c=False)

        @pl.when(i >= _NCH_N)
        def _():
            sum_rows(slot, with_c=True)
        store(i, slot).start()

    fire(0, 0)

    def pair(s, carry):
        fire(2 * s + 1, 1)
        consume(2 * s, 0, s)

        @pl.when(s < _NPAIR - 1)
        def _():
            fire(2 * s + 2, 0)
        consume(2 * s + 1, 1, s)
        return carry

    lax.fori_loop(0, _NPAIR, pair, None)
    store(_NCHUNK - 2, 0).wait()
    store(_NCHUNK - 1, 1).wait()


def _gather_sum(nd_flat, ed_flat, src, dst, emb_table, tbig):
    mesh = plsc.VectorSubcoreMesh(core_axis_name="c", subcore_axis_name="s")
    fn = pl.kernel(
        _sc_body,
        out_type=jax.ShapeDtypeStruct((_B * _T, _D), jnp.float32),
        mesh=mesh,
        scratch_types=[
            pltpu.VMEM((4 * _NNODE,), jnp.int32),
            pltpu.VMEM((4 * _NEDGE,), jnp.int32),
            pltpu.VMEM((2 * _NEDGE,), jnp.int32),
            pltpu.VMEM((2 * _NEDGE,), jnp.int32),
            [pltpu.VMEM((4 * _C, _D), jnp.float32)] * 2,
            [pltpu.VMEM((2 * _C, _D), jnp.float32)] * 2,
            [pltpu.VMEM((_C, _D), jnp.float32)] * 2,
            pltpu.SemaphoreType.DMA,
            [pltpu.SemaphoreType.DMA] * 2,
            [pltpu.SemaphoreType.DMA] * 2,
            [pltpu.SemaphoreType.DMA] * 2,
        ],
    )
    return fn(nd_flat, ed_flat, src, dst, emb_table, tbig)


def kernel(node_data, edge_data, edge_index, lap_eigvec, emb_table, lap_W,
           order_table):
    tbig = _build_table(lap_eigvec, lap_W, order_table)
    feat = _gather_sum(node_data.reshape(-1), edge_data.reshape(-1),
                       edge_index[0], edge_index[1], emb_table, tbig)
    padded_feature = feat.reshape(_B, _T, _D)
    node_part = jnp.broadcast_to(
        jnp.arange(_N, dtype=edge_index.dtype)[None, :, None], (_B, _N, 2))
    edge_part = jnp.transpose(edge_index).reshape(_B, _E, 2)
    padded_index = jnp.concatenate([node_part, edge_part], axis=1)
    padding_mask = jnp.zeros((_B, _T), dtype=jnp.bool_)
    return padded_feature, padding_mask, padded_index
